# baseline (speedup 1.0000x reference)
import jax
import jax.numpy as jnp
from jax.experimental import pallas as pl
from jax.experimental.pallas import tpu as pltpu

_PAIR = 4


def _fused_body(adj_hbm, x_ref, W_ref, b_ref, a_ref, mw_ref, mb_ref,
                mu_ref, sd_ref, buf, sem):
    i = pl.program_id(0)
    nsteps = pl.num_programs(0)

    @pl.when(i == 0)
    def _prologue():
        pltpu.make_async_copy(
            adj_hbm.at[pl.ds(0, _PAIR)], buf.at[0], sem.at[0]).start()

    @pl.when(i + 1 < nsteps)
    def _prefetch():
        nxt = (i + 1) % 2
        pltpu.make_async_copy(
            adj_hbm.at[pl.ds((i + 1) * _PAIR, _PAIR)], buf.at[nxt],
            sem.at[nxt]).start()

    slot = i % 2
    pltpu.make_async_copy(
        adj_hbm.at[pl.ds(i * _PAIR, _PAIR)], buf.at[slot], sem.at[slot]
    ).wait()

    L = W_ref.shape[0]
    Gs = [buf[slot, g, 0].astype(jnp.bfloat16) for g in range(_PAIR)]
    xs = [x_ref[g] for g in range(_PAIR)]
    dn_rhs1 = (((1,), (1,)), ((), ()))
    for li in range(L):
        for g in range(_PAIR):
            mx = jax.lax.dot_general(
                xs[g], W_ref[li, 0], dn_rhs1,
                preferred_element_type=jnp.float32)
            mx = mx + b_ref[li, 0][None, :]
            xo = jnp.dot(Gs[g], mx.astype(jnp.bfloat16),
                         preferred_element_type=jnp.float32)
            a = a_ref[0, li]
            xo = jnp.where(xo >= 0, xo, a * xo)
            xs[g] = xo + xs[g]
    mix = mw_ref.shape[0]
    wmean = jnp.full((mix, 1), 1.0 / mix, dtype=jnp.float32)
    wvar = jnp.full((mix, 1), 1.0 / (mix - 1), dtype=jnp.float32)
    for g in range(_PAIR):
        y = jax.lax.dot_general(
            xs[g], mw_ref[:, 0, :], dn_rhs1,
            preferred_element_type=jnp.float32)
        y = y + mb_ref[:, 0][None, :]
        mu = jnp.dot(y, wmean, preferred_element_type=jnp.float32)
        d = y - mu
        var = jnp.dot(d * d, wvar, preferred_element_type=jnp.float32)
        mu_ref[g] = mu
        sd_ref[g] = jnp.sqrt(var)


def kernel(adj, vect_feat, input_mask, input_idx, adj_oh, gml_W, gml_b,
           gml_prelu, mix_W, mix_b):
    B, GS, N, _ = adj.shape
    F = vect_feat.shape[-1]
    L = gml_W.shape[0]
    MIX, OUT = mix_W.shape[0], mix_W.shape[1]

    adj3 = adj.reshape(B, GS, N, N)
    a2 = gml_prelu.reshape(1, L)

    mu, sd = pl.pallas_call(
        _fused_body,
        grid=(B // _PAIR,),
        in_specs=[
            pl.BlockSpec(memory_space=pltpu.MemorySpace.HBM),
            pl.BlockSpec((_PAIR, N, F), lambda i: (i, 0, 0)),
            pl.BlockSpec((L, GS, F, F), lambda i: (0, 0, 0, 0)),
            pl.BlockSpec((L, GS, F), lambda i: (0, 0, 0)),
            pl.BlockSpec((1, L), lambda i: (0, 0)),
            pl.BlockSpec((MIX, OUT, F), lambda i: (0, 0, 0)),
            pl.BlockSpec((MIX, OUT), lambda i: (0, 0)),
        ],
        out_specs=[
            pl.BlockSpec((_PAIR, N, OUT), lambda i: (i, 0, 0)),
            pl.BlockSpec((_PAIR, N, OUT), lambda i: (i, 0, 0)),
        ],
        out_shape=[
            jax.ShapeDtypeStruct((B, N, OUT), jnp.float32),
            jax.ShapeDtypeStruct((B, N, OUT), jnp.float32),
        ],
        scratch_shapes=[
            pltpu.VMEM((2, _PAIR, GS, N, N), jnp.float32),
            pltpu.SemaphoreType.DMA((2,)),
        ],
        compiler_params=pltpu.CompilerParams(
            dimension_semantics=("arbitrary",),
        ),
    )(adj3, vect_feat, gml_W, gml_b, a2, mix_W, mix_b)

    return mu, sd
